# Initial kernel scaffold; baseline (speedup 1.0000x reference)
import jax
import jax.numpy as jnp
from jax.experimental import pallas as pl


def kernel(pred, gt, heatmap):
    raise NotImplementedError("write your pallas kernel here")


# speedup vs baseline: 2.1464x; 2.1464x over previous
import jax
import jax.numpy as jnp
from jax import lax
from jax.experimental import pallas as pl
from jax.experimental.pallas import tpu as pltpu
from jax.experimental.pallas import tpu_sc as plsc

L = 16
B, J, M, N, D = 32, 64, 128, 128, 11
MN = M * N
CH = 2
WORDS = CH * MN
NCHUNK = J // CH
GD = J * D
GD_PAD = 720


def _body(hm, pred_h, gt_h, out_h, buf_a, buf_b, pv, gv, res_v, sem_a, sem_b):
    w = lax.axis_index("c") * 16 + lax.axis_index("s")

    pltpu.sync_copy(pred_h.at[w], pv.at[pl.ds(0, GD)])
    pltpu.sync_copy(gt_h.at[w], gv.at[pl.ds(0, GD)])
    zeros = jnp.zeros((L,), jnp.float32)
    pv[pl.ds(GD, L)] = zeros
    gv[pl.ds(GD, L)] = zeros

    iota = lax.iota(jnp.int32, L)
    neg_inf = jnp.full((L,), -jnp.inf, jnp.float32)
    cls_mask = iota < 7

    def lane(v, c):
        return jnp.sum(jnp.where(iota == c, v, 0.0))

    def chunk_slice(c):
        return hm.at[w, pl.ds(c * WORDS, WORDS)]

    def process_obj(buf, off, j_obj, acc):
        def ibody(i, carry):
            vmax, vidx, vpos = carry
            v = buf[pl.ds(off + i * L, L)]
            m = v > vmax
            return (jnp.where(m, v, vmax),
                    jnp.where(m, vpos, vidx),
                    vpos + L)

        vmax, vidx, _ = lax.fori_loop(
            0, MN // L, ibody, (neg_inf, iota, iota), unroll=8)

        gmax = jnp.max(vmax)
        cand = jnp.where(vmax == gmax, vidx, jnp.int32(MN))
        flat = jnp.min(cand)
        xf = (flat >> 7).astype(jnp.float32)
        yf = (flat & 127).astype(jnp.float32)

        vg = gv[pl.ds(j_obj * D, L)]
        vp = pv[pl.ds(j_obj * D, L)]
        dd = vp - vg
        cls = jnp.sum(jnp.where(cls_mask, dd * dd, 0.0))
        g7 = lane(vg, 7)
        g8 = lane(vg, 8)
        g9 = lane(vg, 9)
        g10 = lane(vg, 10)
        t1 = g9 + g7 - xf - lane(vp, 7)
        t2 = g10 + g8 - yf - lane(vp, 8)
        conf = (1.0 - gmax) * (1.0 - gmax)
        valid = ((g9 > 0.0) & (g10 > 0.0)
                 & (g9 < float(M)) & (g10 < float(N)))
        per = jnp.where(valid, cls + t1 * t1 + t2 * t2 + conf, 0.0)
        return acc + per

    pltpu.async_copy(chunk_slice(0), buf_a, sem_a)

    def obody(c2, acc):
        c_a = 2 * c2
        c_b = c_a + 1
        pltpu.async_copy(chunk_slice(c_b), buf_b, sem_b)
        pltpu.make_async_copy(chunk_slice(c_a), buf_a, sem_a).wait()
        for k in range(CH):
            acc = process_obj(buf_a, k * MN, c_a * CH + k, acc)

        @pl.when(c2 < NCHUNK // 2 - 1)
        def _():
            pltpu.async_copy(chunk_slice(c_a + 2), buf_a, sem_a)

        pltpu.make_async_copy(chunk_slice(c_b), buf_b, sem_b).wait()
        for k in range(CH):
            acc = process_obj(buf_b, k * MN, c_b * CH + k, acc)
        return acc

    acc = lax.fori_loop(0, NCHUNK // 2, obody, jnp.float32(0.0))
    res_v[...] = jnp.full((L,), acc, jnp.float32)
    pltpu.sync_copy(res_v, out_h.at[w])


@jax.jit
def kernel(pred, gt, heatmap):
    hm2 = heatmap.reshape(B, J * MN)
    p2 = pred.reshape(B, GD)
    g2 = gt.reshape(B, GD)
    mesh = plsc.VectorSubcoreMesh(core_axis_name="c", subcore_axis_name="s")
    out = pl.kernel(
        _body,
        out_type=jax.ShapeDtypeStruct((B, L), jnp.float32),
        mesh=mesh,
        compiler_params=pltpu.CompilerParams(
            needs_layout_passes=False, use_tc_tiling_on_sc=False),
        scratch_types=[
            pltpu.VMEM((WORDS,), jnp.float32),
            pltpu.VMEM((WORDS,), jnp.float32),
            pltpu.VMEM((GD_PAD,), jnp.float32),
            pltpu.VMEM((GD_PAD,), jnp.float32),
            pltpu.VMEM((L,), jnp.float32),
            pltpu.SemaphoreType.DMA,
            pltpu.SemaphoreType.DMA,
        ],
    )(hm2, p2, g2)
    return out[:, 0]
